# Initial kernel scaffold; baseline (speedup 1.0000x reference)
import jax
import jax.numpy as jnp
from jax.experimental import pallas as pl


def kernel(scores_map, descriptor_map):
    raise NotImplementedError("write your pallas kernel here")


# speedup vs baseline: 2.1901x; 2.1901x over previous
import jax
import jax.numpy as jnp
from jax.experimental import pallas as pl
from jax.experimental.pallas import tpu as pltpu

_RADIUS = 2
_TOP_K = 500
_KERNEL = 4
_H = 1024
_W = 1024
_C = 64


def _detect_kernel(s_ref, v_ref, r_ref, c_ref):
    s = s_ref[...]
    row = jax.lax.broadcasted_iota(jnp.int32, (_H, _W), 0)
    col = jax.lax.broadcasted_iota(jnp.int32, (_H, _W), 1)
    live = (
        (row > _RADIUS)
        & (row < _H - _RADIUS)
        & (col > _RADIUS)
        & (col < _W - _RADIUS)
    )
    s = jnp.where(live, s, 0.0)

    nh = _H // _KERNEL
    nw = _W // _KERNEL

    s3 = s.reshape(nh, _KERNEL, _W)
    c_local = jax.lax.broadcasted_iota(jnp.int32, (nh, _W), 1) % _KERNEL
    c_localf = c_local.astype(jnp.float32)
    v = s3[:, 0, :]
    l = c_localf
    for r in range(1, _KERNEL):
        sv = s3[:, r, :]
        lr = c_localf + float(r * _KERNEL)
        take = sv > v
        v = jnp.where(take, sv, v)
        l = jnp.where(take, lr, l)

    vt = v.T.reshape(nw, _KERNEL, nh)
    lt = l.T.reshape(nw, _KERNEL, nh)
    V = vt[:, 0, :]
    L = lt[:, 0, :]
    for c in range(1, _KERNEL):
        cv = vt[:, c, :]
        cl = lt[:, c, :]
        take = (cv > V) | ((cv == V) & (cl < L))
        V = jnp.where(take, cv, V)
        L = jnp.where(take, cl, L)
    V = V.T
    L = L.T.astype(jnp.int32)

    tile_r = jax.lax.broadcasted_iota(jnp.int32, (nh, nw), 0)
    tile_c = jax.lax.broadcasted_iota(jnp.int32, (nh, nw), 1)
    v_ref[...] = V
    r_ref[...] = tile_r * _KERNEL + L // _KERNEL
    c_ref[...] = tile_c * _KERNEL + L % _KERNEL


_NBUF = 8
_LANES = 128


def _gather_kernel(kp_ref, d_hbm, out_ref, buf, sems):
    def issue(i, slot):
        y = kp_ref[i, 1]
        x_blk = (kp_ref[i, 0] // _LANES) * _LANES
        pltpu.make_async_copy(
            d_hbm.at[0, :, pl.ds(y, 1), pl.ds(x_blk, _LANES)],
            buf.at[slot],
            sems.at[slot],
        ).start()

    for s in range(_NBUF):
        issue(s, s)

    lane = jax.lax.broadcasted_iota(jnp.int32, (1, _LANES), 1)

    def loop(i, carry):
        slot = jax.lax.rem(i, _NBUF)
        y = kp_ref[i, 1]
        x_blk = (kp_ref[i, 0] // _LANES) * _LANES
        pltpu.make_async_copy(
            d_hbm.at[0, :, pl.ds(y, 1), pl.ds(x_blk, _LANES)],
            buf.at[slot],
            sems.at[slot],
        ).wait()
        x_in = kp_ref[i, 0] - x_blk
        sel = (lane == x_in).astype(jnp.float32)
        d = jnp.sum(buf[slot, :, 0, :] * sel, axis=1)
        norm = jnp.sqrt(jnp.sum(d * d))
        out_ref[pl.ds(i, 1), :] = (d / norm).reshape(1, _C)

        @pl.when(i + _NBUF < _TOP_K)
        def _():
            issue(i + _NBUF, slot)

        return carry

    jax.lax.fori_loop(0, _TOP_K, loop, 0)


def kernel(scores_map, descriptor_map):
    nh = _H // _KERNEL
    nw = _W // _KERNEL
    V, R, Cc = pl.pallas_call(
        _detect_kernel,
        out_shape=(
            jax.ShapeDtypeStruct((nh, nw), jnp.float32),
            jax.ShapeDtypeStruct((nh, nw), jnp.int32),
            jax.ShapeDtypeStruct((nh, nw), jnp.int32),
        ),
    )(scores_map[0, 0])

    vals = V.ravel()
    flat_indices = jnp.argsort(vals)[-_TOP_K:]
    top_values = vals[flat_indices]
    top_rows = R.ravel()[flat_indices]
    top_cols = Cc.ravel()[flat_indices]
    keypoints = jnp.stack([top_cols, top_rows], axis=1)

    desc = pl.pallas_call(
        _gather_kernel,
        grid_spec=pltpu.PrefetchScalarGridSpec(
            num_scalar_prefetch=1,
            grid=(1,),
            in_specs=[pl.BlockSpec(memory_space=pl.ANY)],
            out_specs=pl.BlockSpec((_TOP_K, _C), lambda i, kp: (0, 0)),
            scratch_shapes=[
                pltpu.VMEM((_NBUF, _C, 1, _LANES), jnp.float32),
                pltpu.SemaphoreType.DMA((_NBUF,)),
            ],
        ),
        out_shape=jax.ShapeDtypeStruct((_TOP_K, _C), jnp.float32),
    )(keypoints, descriptor_map)

    return (keypoints, desc[:, :, None], top_values)


# speedup vs baseline: 2.7983x; 1.2777x over previous
import jax
import jax.numpy as jnp
from jax.experimental import pallas as pl
from jax.experimental.pallas import tpu as pltpu

_RADIUS = 2
_TOP_K = 500
_KERNEL = 4
_H = 1024
_W = 1024
_C = 64


def _detect_kernel(s_ref, v_ref):
    s = s_ref[...]
    row = jax.lax.broadcasted_iota(jnp.int32, (_H, _W), 0)
    col = jax.lax.broadcasted_iota(jnp.int32, (_H, _W), 1)
    live = (
        (row > _RADIUS)
        & (row < _H - _RADIUS)
        & (col > _RADIUS)
        & (col < _W - _RADIUS)
    )
    s = jnp.where(live, s, 0.0)

    nh = _H // _KERNEL
    nw = _W // _KERNEL

    s3 = s.reshape(nh, _KERNEL, _W)
    c_local = jax.lax.broadcasted_iota(jnp.int32, (nh, _W), 1) % _KERNEL
    c_localf = c_local.astype(jnp.float32)
    v = s3[:, 0, :]
    l = c_localf
    for r in range(1, _KERNEL):
        sv = s3[:, r, :]
        lr = c_localf + float(r * _KERNEL)
        take = sv > v
        v = jnp.where(take, sv, v)
        l = jnp.where(take, lr, l)

    vt = v.T.reshape(nw, _KERNEL, nh)
    lt = l.T.reshape(nw, _KERNEL, nh)
    V = vt[:, 0, :]
    L = lt[:, 0, :]
    for c in range(1, _KERNEL):
        cv = vt[:, c, :]
        cl = lt[:, c, :]
        take = (cv > V) | ((cv == V) & (cl < L))
        V = jnp.where(take, cv, V)
        L = jnp.where(take, cl, L)
    V = V.T
    L = L.T.astype(jnp.int32)

    nt = nh * nw
    S = 512
    b = jax.lax.bitcast_convert_type(V, jnp.int32)
    idx = jax.lax.broadcasted_iota(jnp.int32, (nh, nw), 0) * nw + \
        jax.lax.broadcasted_iota(jnp.int32, (nh, nw), 1)

    def _vstep(_, lohi):
        lo, hi = lohi
        mid = (lo + hi) // 2
        cnt = jnp.sum((b > mid).astype(jnp.int32))
        small = cnt < _TOP_K
        return (jnp.where(small, lo, mid), jnp.where(small, mid, hi))

    _, vt = jax.lax.fori_loop(0, 32, _vstep, (jnp.int32(-1), jnp.int32(2139095040)))
    n1 = jnp.sum((b > vt).astype(jnp.int32))
    t_need = _TOP_K - n1
    ties = b == vt

    def _istep(_, lohi):
        lo, hi = lohi
        mid = (lo + hi) // 2
        cnt = jnp.sum((ties & (idx > mid)).astype(jnp.int32))
        small = cnt < t_need
        return (jnp.where(small, lo, mid), jnp.where(small, mid, hi))

    _, jt = jax.lax.fori_loop(0, 18, _istep, (jnp.int32(-1), jnp.int32(nt)))
    P = (b > vt) | (ties & (idx >= jt))
    Pf = P.astype(jnp.float32)

    lane = jax.lax.broadcasted_iota(jnp.int32, (nh, nw), 1)
    q = Pf
    for sh in (1, 2, 4, 8, 16, 32, 64, 128):
        q = q + jnp.where(lane >= sh, jnp.roll(q, sh, axis=1), 0.0)
    n_row = q[:, nw - 1 : nw].T
    lane_s = jax.lax.broadcasted_iota(jnp.int32, (1, nh), 1)
    ro = n_row
    for sh in (1, 2, 4, 8, 16, 32, 64, 128):
        ro = ro + jnp.where(lane_s >= sh, jnp.roll(ro, sh, axis=1), 0.0)
    row_off = ro - n_row

    s_col = jax.lax.broadcasted_iota(jnp.int32, (S, nh), 0).astype(jnp.float32)
    R1h = ((row_off <= s_col) & (s_col < row_off + n_row)).astype(jnp.float32)
    rowoff_s = jnp.sum(R1h * row_off, axis=1, keepdims=True)
    s_col1 = jax.lax.broadcasted_iota(jnp.int32, (S, 1), 0).astype(jnp.float32)
    k_s = s_col1 - rowoff_s
    qm = jnp.where(P, q, 0.0)
    Qm = jnp.dot(R1h, qm, preferred_element_type=jnp.float32, precision=jax.lax.Precision.HIGHEST)
    C1h = (Qm == k_s + 1.0).astype(jnp.float32)

    Vs = jnp.dot(R1h, V, preferred_element_type=jnp.float32, precision=jax.lax.Precision.HIGHEST)
    Lf = L.astype(jnp.float32)
    Ls = jnp.dot(R1h, Lf, preferred_element_type=jnp.float32, precision=jax.lax.Precision.HIGHEST)
    lane_r = jax.lax.broadcasted_iota(jnp.int32, (S, nh), 1).astype(jnp.float32)
    lane_c = jax.lax.broadcasted_iota(jnp.int32, (S, nw), 1).astype(jnp.float32)
    v_s = jnp.sum(Vs * C1h, axis=1, keepdims=True)
    l_s = jnp.sum(Ls * C1h, axis=1, keepdims=True)
    r_s = jnp.sum(R1h * lane_r, axis=1, keepdims=True)
    c_s = jnp.sum(C1h * lane_c, axis=1, keepdims=True)
    pad = s_col1 >= float(_TOP_K)
    v_s = jnp.where(pad, 2.0, v_s)
    i_s = r_s * float(nw) + c_s

    v_row = v_s.T
    i_row = i_s.T
    less = (v_row < v_s) | ((v_row == v_s) & (i_row < i_s))
    rank = jnp.sum(less.astype(jnp.float32), axis=1, keepdims=True)
    F = (rank.T == jax.lax.broadcasted_iota(jnp.int32, (S, S), 0).astype(jnp.float32)).astype(jnp.float32)
    v_t = jnp.sum(F * v_row, axis=1, keepdims=True)
    r_t = jnp.sum(F * r_s.T, axis=1, keepdims=True)
    c_t = jnp.sum(F * c_s.T, axis=1, keepdims=True)
    l_t = jnp.sum(F * l_s.T, axis=1, keepdims=True)

    gr_t = r_t * float(_KERNEL) + jnp.floor(l_t / float(_KERNEL))
    gc_t = c_t * float(_KERNEL) + (l_t - jnp.floor(l_t / float(_KERNEL)) * float(_KERNEL))

    out_lane = jax.lax.broadcasted_iota(jnp.int32, (S, 128), 1)
    out = jnp.where(out_lane == 0, v_t, 0.0)
    out = jnp.where(out_lane == 1, gr_t, out)
    out = jnp.where(out_lane == 2, gc_t, out)
    v_ref[...] = out


_NBUF = 8
_LANES = 128


def _gather_kernel(kp_ref, d_hbm, out_ref, buf, sems):
    def issue(i, slot):
        y = kp_ref[i, 1]
        x_blk = (kp_ref[i, 0] // _LANES) * _LANES
        pltpu.make_async_copy(
            d_hbm.at[0, :, pl.ds(y, 1), pl.ds(x_blk, _LANES)],
            buf.at[slot],
            sems.at[slot],
        ).start()

    for s in range(_NBUF):
        issue(s, s)

    lane = jax.lax.broadcasted_iota(jnp.int32, (1, _LANES), 1)

    def loop(i, carry):
        slot = jax.lax.rem(i, _NBUF)
        y = kp_ref[i, 1]
        x_blk = (kp_ref[i, 0] // _LANES) * _LANES
        pltpu.make_async_copy(
            d_hbm.at[0, :, pl.ds(y, 1), pl.ds(x_blk, _LANES)],
            buf.at[slot],
            sems.at[slot],
        ).wait()
        x_in = kp_ref[i, 0] - x_blk
        sel = (lane == x_in).astype(jnp.float32)
        d = jnp.sum(buf[slot, :, 0, :] * sel, axis=1)
        norm = jnp.sqrt(jnp.sum(d * d))
        out_ref[pl.ds(i, 1), :] = (d / norm).reshape(1, _C)

        @pl.when(i + _NBUF < _TOP_K)
        def _():
            issue(i + _NBUF, slot)

        return carry

    jax.lax.fori_loop(0, _TOP_K, loop, 0)


def kernel(scores_map, descriptor_map):
    topk = pl.pallas_call(
        _detect_kernel,
        out_shape=jax.ShapeDtypeStruct((512, 128), jnp.float32),
    )(scores_map[0, 0])

    top_values = topk[: _TOP_K, 0]
    top_rows = topk[: _TOP_K, 1].astype(jnp.int32)
    top_cols = topk[: _TOP_K, 2].astype(jnp.int32)
    keypoints = jnp.stack([top_cols, top_rows], axis=1)

    desc = pl.pallas_call(
        _gather_kernel,
        grid_spec=pltpu.PrefetchScalarGridSpec(
            num_scalar_prefetch=1,
            grid=(1,),
            in_specs=[pl.BlockSpec(memory_space=pl.ANY)],
            out_specs=pl.BlockSpec((_TOP_K, _C), lambda i, kp: (0, 0)),
            scratch_shapes=[
                pltpu.VMEM((_NBUF, _C, 1, _LANES), jnp.float32),
                pltpu.SemaphoreType.DMA((_NBUF,)),
            ],
        ),
        out_shape=jax.ShapeDtypeStruct((_TOP_K, _C), jnp.float32),
    )(keypoints, descriptor_map)

    return (keypoints, desc[:, :, None], top_values)


# speedup vs baseline: 5.9038x; 2.1098x over previous
import jax
import jax.numpy as jnp
from jax.experimental import pallas as pl
from jax.experimental.pallas import tpu as pltpu

_RADIUS = 2
_TOP_K = 500
_KERNEL = 4
_H = 1024
_W = 1024
_C = 64


def _detect_kernel(s_ref, v_ref):
    s = s_ref[...]
    row = jax.lax.broadcasted_iota(jnp.int32, (_H, _W), 0)
    col = jax.lax.broadcasted_iota(jnp.int32, (_H, _W), 1)
    live = (
        (row > _RADIUS)
        & (row < _H - _RADIUS)
        & (col > _RADIUS)
        & (col < _W - _RADIUS)
    )
    s = jnp.where(live, s, 0.0)

    nh = _H // _KERNEL
    nw = _W // _KERNEL

    s3 = s.reshape(nh, _KERNEL, _W)
    c_local = jax.lax.broadcasted_iota(jnp.int32, (nh, _W), 1) % _KERNEL
    c_localf = c_local.astype(jnp.float32)
    v = s3[:, 0, :]
    l = c_localf
    for r in range(1, _KERNEL):
        sv = s3[:, r, :]
        lr = c_localf + float(r * _KERNEL)
        take = sv > v
        v = jnp.where(take, sv, v)
        l = jnp.where(take, lr, l)

    vt = v.T.reshape(nw, _KERNEL, nh)
    lt = l.T.reshape(nw, _KERNEL, nh)
    V = vt[:, 0, :]
    L = lt[:, 0, :]
    for c in range(1, _KERNEL):
        cv = vt[:, c, :]
        cl = lt[:, c, :]
        take = (cv > V) | ((cv == V) & (cl < L))
        V = jnp.where(take, cv, V)
        L = jnp.where(take, cl, L)
    V = V.T
    L = L.T.astype(jnp.int32)

    nt = nh * nw
    S = 512
    b = jax.lax.bitcast_convert_type(V, jnp.int32)
    idx = jax.lax.broadcasted_iota(jnp.int32, (nh, nw), 0) * nw + \
        jax.lax.broadcasted_iota(jnp.int32, (nh, nw), 1)

    def _vstep(_, lohi):
        lo, hi = lohi
        mid = (lo + hi) // 2
        cnt = jnp.sum((b > mid).astype(jnp.int32))
        small = cnt < _TOP_K
        return (jnp.where(small, lo, mid), jnp.where(small, mid, hi))

    _, vt = jax.lax.fori_loop(0, 32, _vstep, (jnp.int32(-1), jnp.int32(2139095040)))
    n1 = jnp.sum((b > vt).astype(jnp.int32))
    t_need = _TOP_K - n1
    ties = b == vt

    def _istep(_, lohi):
        lo, hi = lohi
        mid = (lo + hi) // 2
        cnt = jnp.sum((ties & (idx > mid)).astype(jnp.int32))
        small = cnt < t_need
        return (jnp.where(small, lo, mid), jnp.where(small, mid, hi))

    _, jt = jax.lax.fori_loop(0, 18, _istep, (jnp.int32(-1), jnp.int32(nt)))
    P = (b > vt) | (ties & (idx >= jt))
    Pf = P.astype(jnp.float32)

    lane = jax.lax.broadcasted_iota(jnp.int32, (nh, nw), 1)
    q = Pf
    for sh in (1, 2, 4, 8, 16, 32, 64, 128):
        q = q + jnp.where(lane >= sh, jnp.roll(q, sh, axis=1), 0.0)
    n_row = q[:, nw - 1 : nw].T
    lane_s = jax.lax.broadcasted_iota(jnp.int32, (1, nh), 1)
    ro = n_row
    for sh in (1, 2, 4, 8, 16, 32, 64, 128):
        ro = ro + jnp.where(lane_s >= sh, jnp.roll(ro, sh, axis=1), 0.0)
    row_off = ro - n_row

    s_col = jax.lax.broadcasted_iota(jnp.int32, (S, nh), 0).astype(jnp.float32)
    R1h = ((row_off <= s_col) & (s_col < row_off + n_row)).astype(jnp.float32)
    rowoff_s = jnp.sum(R1h * row_off, axis=1, keepdims=True)
    s_col1 = jax.lax.broadcasted_iota(jnp.int32, (S, 1), 0).astype(jnp.float32)
    k_s = s_col1 - rowoff_s
    qm = jnp.where(P, q, 0.0)
    Qm = jnp.dot(R1h, qm, preferred_element_type=jnp.float32, precision=jax.lax.Precision.HIGHEST)
    C1h = (Qm == k_s + 1.0).astype(jnp.float32)

    Vs = jnp.dot(R1h, V, preferred_element_type=jnp.float32, precision=jax.lax.Precision.HIGHEST)
    Lf = L.astype(jnp.float32)
    Ls = jnp.dot(R1h, Lf, preferred_element_type=jnp.float32, precision=jax.lax.Precision.HIGHEST)
    lane_r = jax.lax.broadcasted_iota(jnp.int32, (S, nh), 1).astype(jnp.float32)
    lane_c = jax.lax.broadcasted_iota(jnp.int32, (S, nw), 1).astype(jnp.float32)
    v_s = jnp.sum(Vs * C1h, axis=1, keepdims=True)
    l_s = jnp.sum(Ls * C1h, axis=1, keepdims=True)
    r_s = jnp.sum(R1h * lane_r, axis=1, keepdims=True)
    c_s = jnp.sum(C1h * lane_c, axis=1, keepdims=True)
    pad = s_col1 >= float(_TOP_K)
    v_s = jnp.where(pad, 2.0, v_s)
    i_s = r_s * float(nw) + c_s

    v_row = v_s.T
    i_row = i_s.T
    less = (v_row < v_s) | ((v_row == v_s) & (i_row < i_s))
    rank = jnp.sum(less.astype(jnp.float32), axis=1, keepdims=True)
    F = (rank.T == jax.lax.broadcasted_iota(jnp.int32, (S, S), 0).astype(jnp.float32)).astype(jnp.float32)
    v_t = jnp.sum(F * v_row, axis=1, keepdims=True)
    r_t = jnp.sum(F * r_s.T, axis=1, keepdims=True)
    c_t = jnp.sum(F * c_s.T, axis=1, keepdims=True)
    l_t = jnp.sum(F * l_s.T, axis=1, keepdims=True)

    gr_t = r_t * float(_KERNEL) + jnp.floor(l_t / float(_KERNEL))
    gc_t = c_t * float(_KERNEL) + (l_t - jnp.floor(l_t / float(_KERNEL)) * float(_KERNEL))

    out_lane = jax.lax.broadcasted_iota(jnp.int32, (S, 128), 1)
    out = jnp.where(out_lane == 0, v_t, 0.0)
    out = jnp.where(out_lane == 1, gr_t, out)
    out = jnp.where(out_lane == 2, gc_t, out)
    v_ref[...] = out


_LANES = 128
_GRP = 8
_NGRP = (_TOP_K + _GRP - 1) // _GRP


def _gather_kernel(kp_ref, d_hbm, out_ref, buf, sel_ref, sems):
    lane = jax.lax.broadcasted_iota(jnp.int32, (1, _LANES), 1)

    def issue_group(g, slot):
        for j in range(_GRP):
            i = jnp.minimum(g * _GRP + j, _TOP_K - 1)
            y = kp_ref[i, 1]
            x_blk = (kp_ref[i, 0] // _LANES) * _LANES
            pltpu.make_async_copy(
                d_hbm.at[0, :, pl.ds(y, 1), pl.ds(x_blk, _LANES)],
                buf.at[slot, j],
                sems.at[slot, j],
            ).start()

    issue_group(0, 0)
    issue_group(1, 1)

    def loop(g, carry):
        slot = jax.lax.rem(g, 2)
        for j in range(_GRP):
            i = jnp.minimum(g * _GRP + j, _TOP_K - 1)
            y = kp_ref[i, 1]
            x_blk = (kp_ref[i, 0] // _LANES) * _LANES
            pltpu.make_async_copy(
                d_hbm.at[0, :, pl.ds(y, 1), pl.ds(x_blk, _LANES)],
                buf.at[slot, j],
                sems.at[slot, j],
            ).wait()
            x_in = kp_ref[i, 0] - x_blk
            sel_ref[j : j + 1, :] = (lane == x_in).astype(jnp.float32)

        sel = sel_ref[...].reshape(_GRP, 1, _LANES)
        d = jnp.sum(buf[slot, :, :, 0, :] * sel, axis=2)
        norm = jnp.sqrt(jnp.sum(d * d, axis=1, keepdims=True))
        out_ref[pl.ds(g * _GRP, _GRP), :] = d / norm

        @pl.when(g + 2 < _NGRP)
        def _():
            issue_group(g + 2, slot)

        return carry

    jax.lax.fori_loop(0, _NGRP, loop, 0)


def kernel(scores_map, descriptor_map):
    topk = pl.pallas_call(
        _detect_kernel,
        out_shape=jax.ShapeDtypeStruct((512, 128), jnp.float32),
    )(scores_map[0, 0])

    top_values = topk[: _TOP_K, 0]
    top_rows = topk[: _TOP_K, 1].astype(jnp.int32)
    top_cols = topk[: _TOP_K, 2].astype(jnp.int32)
    keypoints = jnp.stack([top_cols, top_rows], axis=1)

    desc = pl.pallas_call(
        _gather_kernel,
        grid_spec=pltpu.PrefetchScalarGridSpec(
            num_scalar_prefetch=1,
            grid=(1,),
            in_specs=[pl.BlockSpec(memory_space=pl.ANY)],
            out_specs=pl.BlockSpec((_NGRP * _GRP, _C), lambda i, kp: (0, 0)),
            scratch_shapes=[
                pltpu.VMEM((2, _GRP, _C, 1, _LANES), jnp.float32),
                pltpu.VMEM((_GRP, _LANES), jnp.float32),
                pltpu.SemaphoreType.DMA((2, _GRP)),
            ],
        ),
        out_shape=jax.ShapeDtypeStruct((_NGRP * _GRP, _C), jnp.float32),
    )(keypoints, descriptor_map)

    return (keypoints, desc[:_TOP_K, :, None], top_values)
